# baseline (speedup 1.0000x reference)
import functools

import jax
import jax.numpy as jnp
from jax import lax
from jax.experimental import pallas as pl
from jax.experimental.pallas import tpu as pltpu
from jax.experimental.pallas import tpu_sc as plsc

N = 10000
NP = 10240
D = 256
E = 160000
B = 4096
NT = 16
NC = 2
CH = 128
EPT_RAW = E // NT
NCHUNK = 80
EPT = NCHUNK * CH
DUMMY = NP - 1
NR = 16
NRW = 10
WSZ = NP * NRW
RPT = NP // NT
BLK = 256
GRID = NP // BLK
F32 = jnp.float32


def _tc1_body(ue, ie, sgW, ugW, igW, sgb, ugb, igb, apack, abrow, rpad, abot2,
              hsg, hiug, huig, scal, rout):
    i = pl.program_id(0)
    hp = dict(preferred_element_type=F32, precision=lax.Precision.HIGHEST)
    xu = ue[...]
    xi = ie[...]
    hs = jnp.dot(xu, sgW[...], **hp) + sgb[...]
    hui = jnp.dot(xu, ugW[...], **hp) + ugb[...]
    hii = jnp.dot(xi, ugW[...], **hp) + ugb[...]
    hgu = jnp.dot(xu, igW[...], **hp) + igb[...]
    hgi = jnp.dot(xi, igW[...], **hp) + igb[...]
    hsg[0] = hs[:, :128]
    hsg[1] = hs[:, 128:]
    hiug[0] = hii[:, :128]
    hiug[1] = hii[:, 128:]
    huig[0] = hgu[:, :128]
    huig[1] = hgu[:, 128:]
    scal[...] = (jnp.dot(hs, apack[0], **hp) + jnp.dot(hui, apack[1], **hp)
                 + jnp.dot(hii, apack[2], **hp) + jnp.dot(hgi, apack[3], **hp)
                 + jnp.dot(hgu, apack[4], **hp) + abrow[...])

    @pl.when(i == 0)
    def _():
        rout[...] = jnp.dot(rpad[...], abot2[...], **hp)


def _tc2_body(usr, uhr, ihr, ier, wugr, wigr, denr, ufW, ufb, ifW, ifb,
              rpadr, p1a, p1c, p1b, pu, pi):
    hp = dict(preferred_element_type=F32, precision=lax.Precision.HIGHEST)
    dd = denr[...]
    inv_sg = 1.0 / (dd[:, 0:1] + 1e-16)
    inv_ug = 1.0 / (jnp.sum(wugr[...], axis=1, keepdims=True) + 1e-16)
    inv_ig = 1.0 / (jnp.sum(wigr[...], axis=1, keepdims=True) + 1e-16)
    us = jnp.concatenate([usr[0], usr[1]], axis=1) * inv_sg
    uh = (jnp.concatenate([uhr[0], uhr[1]], axis=1)
          + jnp.dot(wugr[...], rpadr[...], **hp)) * inv_ug
    ucat = jnp.concatenate([us, uh], axis=1)
    fu = jnp.maximum(jnp.dot(ucat, ufW[...], **hp) + ufb[...], 0.0)
    pu[...] = jnp.dot(fu, p1a[...], **hp) + p1b[...]
    ih = (jnp.concatenate([ihr[0], ihr[1]], axis=1)
          + jnp.dot(wigr[...], rpadr[...], **hp)) * inv_ig
    icat = jnp.concatenate([ier[...], ih], axis=1)
    fi = jnp.maximum(jnp.dot(icat, ifW[...], **hp) + ifb[...], 0.0)
    pi[...] = jnp.dot(fi, p1c[...], **hp)


def _sc1_body(comb_sg, comb_ug, comb_ig,
              hsg2, hiug2, huig2,
              dsg, ssg, dug, sug, dig, sig, rau, rai, z1d,
              usagg, uhagg, ihagg, wug, wig, dnsg,
              acc, den, wsp, dsp, ssp,
              ebufA, exbA, wixA, dgbA, sgbA, rbufA,
              ebufB, exbB, wixB, dgbB, sgbB, rbufB,
              rtt, srA, sdA, ssA, swA, srB, sdB, ssB, swB):
    c = lax.axis_index("c")
    t = lax.axis_index("s")
    r0 = t * RPT

    def run_gat(comb, d_hbm, s_hbm, tbl, ratt_hbm, out_hbm, den_hbm, w_hbm,
                use_rating, aux_core):
        def zrow(r, carry2):
            for m in range(8):
                rbufA[r, pl.ds(m * 16, 16)] = jnp.zeros((16,), F32)
            return carry2

        lax.fori_loop(0, CH, zrow, 0)
        for kk in range(RPT // CH):
            pltpu.sync_copy(rbufA, acc.at[pl.ds(r0 + kk * CH, CH)])
        if use_rating:
            pltpu.sync_copy(z1d.at[pl.ds(t * RPT * NRW, RPT * NRW)],
                            wsp.at[pl.ds(t * RPT * NRW, RPT * NRW)])
            pltpu.sync_copy(ratt_hbm, rtt)
        else:
            pltpu.sync_copy(z1d.at[pl.ds(r0, RPT)], den.at[pl.ds(r0, RPT)])
        pltpu.sync_copy(d_hbm.at[pl.ds(r0, RPT)], dsp.at[pl.ds(r0, RPT)])
        pltpu.sync_copy(s_hbm.at[pl.ds(r0, RPT)], ssp.at[pl.ds(r0, RPT)])
        plsc.subcore_barrier()

        def halfchunk(j, eb, ex, wx, dg, sg2, rb, sr, sd, ss):
            pltpu.sync_copy(comb.at[t, j], eb)
            cpd = pltpu.async_copy(dsp.at[eb.at[1]], dg, sd)
            cps = pltpu.async_copy(ssp.at[eb.at[0]], sg2, ss)
            cpr = pltpu.async_copy(tbl.at[c].at[eb.at[0]], rb, sr)
            return cpd, cps, cpr

        def process(eb, ex, wx, dg, sg2, rb, cpd, cps, cpr, sw):
            cpd.wait()
            cps.wait()
            for k in range(CH // 16):
                sl = pl.ds(k * 16, 16)
                e = dg[sl] + sg2[sl]
                if use_rating:
                    rr = plsc.bitcast(eb[2, sl], F32)
                    ri = jnp.clip((rr * 2.0 - 1.0).astype(jnp.int32), 0, 9)
                    e = e + plsc.load_gather(rtt, [ri])
                    wx[sl] = eb[1, sl] * NRW + ri
                e = jnp.where(e >= 0.0, e, 0.2 * e)
                ex[sl] = jnp.exp(e)
            @pl.when(c == aux_core)
            def _():
                if use_rating:
                    pltpu.sync_copy(ex, wsp.at[wx], add=True)
                else:
                    pltpu.sync_copy(ex, den.at[eb.at[1]], add=True)
            cpr.wait()

            def rows(g, carry2):
                exv = ex[pl.ds(g * 16, 16)]
                for q in range(16):
                    s = exv[q]
                    r = g * 16 + q
                    for m in range(8):
                        msl = pl.ds(m * 16, 16)
                        rb[r, msl] = rb[r, msl] * s
                return carry2

            lax.fori_loop(0, CH // 16, rows, 0)
            return pltpu.async_copy(rb, acc.at[eb.at[1]], add=True, sem=sw)

        def chunk(jj, carry):
            j = jj * 2
            hA = halfchunk(j, ebufA, exbA, wixA, dgbA, sgbA, rbufA,
                           srA, sdA, ssA)
            hB = halfchunk(j + 1, ebufB, exbB, wixB, dgbB, sgbB, rbufB,
                           srB, sdB, ssB)
            wA = process(ebufA, exbA, wixA, dgbA, sgbA, rbufA, *hA, swA)
            wB = process(ebufB, exbB, wixB, dgbB, sgbB, rbufB, *hB, swB)
            wA.wait()
            wB.wait()
            return carry

        lax.fori_loop(0, NCHUNK // 2, chunk, 0)
        plsc.subcore_barrier()

        pltpu.sync_copy(acc.at[pl.ds(r0, RPT)], out_hbm.at[c, pl.ds(r0, RPT)])
        if use_rating:
            @pl.when(c == aux_core)
            def _():
                pltpu.sync_copy(wsp.at[pl.ds(t * RPT * NRW, RPT * NRW)],
                                w_hbm.at[pl.ds(t * RPT * NRW, RPT * NRW)])
        else:
            @pl.when(c == aux_core)
            def _():
                pltpu.sync_copy(den.at[pl.ds(r0, RPT)],
                                den_hbm.at[pl.ds(r0, RPT)])

    run_gat(comb_sg, dsg, ssg, hsg2, None, usagg, dnsg, None, False, 1)
    run_gat(comb_ug, dug, sug, hiug2, rau, uhagg, None, wug, True, 0)
    run_gat(comb_ig, dig, sig, huig2, rai, ihagg, None, wig, True, 1)


def _sc2_body(uid, iid, Pu, Pi, p2v, p2s, score, uv, iv, pur, pir, p2b, p2bs,
              ob, sem):
    c = lax.axis_index("c")
    t = lax.axis_index("s")
    w = t * NC + c
    nb = B // (NC * NT)
    base = w * nb
    pltpu.sync_copy(uid.at[pl.ds(base, nb)], uv)
    pltpu.sync_copy(iid.at[pl.ds(base, nb)], iv)
    pltpu.async_copy(Pu.at[uv], pur, sem).wait()
    pltpu.async_copy(Pi.at[iv], pir, sem).wait()
    pltpu.sync_copy(p2v, p2b)
    pltpu.sync_copy(p2s, p2bs)
    p2lo = p2b[pl.ds(0, 16)]
    p2hi = p2b[pl.ds(16, 16)]
    bias = p2bs[...][0]

    def grp(g, carry):
        rows16 = lax.iota(jnp.int32, 16) + g * 16
        acc16 = jnp.zeros((16,), F32)
        for j in range(32):
            cj = jnp.full((16,), j, jnp.int32)
            v = (plsc.load_gather(pur, [rows16, cj])
                 + plsc.load_gather(pir, [rows16, cj]))
            p2j = p2lo[j] if j < 16 else p2hi[j - 16]
            acc16 = acc16 + jnp.maximum(v, 0.0) * p2j
        ob[pl.ds(g * 16, 16)] = acc16 + bias
        return carry

    lax.fori_loop(0, nb // 16, grp, 0)
    pltpu.sync_copy(ob, score.at[pl.ds(base, nb)])


def kernel(user_ids, item_ids, social_adj, interact_adj, interact_ratings,
           user_emb, item_emb, rating_emb,
           sg_W, sg_b, sg_a, sg_ab,
           ug_W, ug_b, ug_a, ug_ab,
           ig_W, ig_b, ig_a, ig_ab,
           uf_W, uf_b, if_W, if_b,
           p1_W, p1_b, p2_W, p2_b):
    ue = jnp.pad(user_emb, ((0, NP - N), (0, 0)))
    ie = jnp.pad(item_emb, ((0, NP - N), (0, 0)))

    def prep_idx(x, fill):
        x = x.reshape(NT, EPT_RAW)
        x = jnp.pad(x, ((0, 0), (0, EPT - EPT_RAW)), constant_values=fill)
        return x.reshape(NT, NCHUNK, CH)

    sa0p = prep_idx(social_adj[0], DUMMY)
    sa1p = prep_idx(social_adj[1], DUMMY)
    ia0p = prep_idx(interact_adj[0], DUMMY)
    ia1p = prep_idx(interact_adj[1], DUMMY)
    ratb = lax.bitcast_convert_type(prep_idx(interact_ratings, 1.0),
                                    jnp.int32)
    comb_sg = jnp.stack([sa0p, sa1p, jnp.zeros_like(sa0p)], axis=2)
    comb_ug = jnp.stack([ia1p, ia0p, ratb], axis=2)
    comb_ig = jnp.stack([ia0p, ia1p, ratb], axis=2)

    def col2(v0, v1):
        z = jnp.zeros((D, 128), F32)
        return z.at[:, 0].set(v0).at[:, 1].set(v1)

    apack = jnp.stack([
        col2(sg_a[:D, 0], sg_a[D:, 0]),
        jnp.zeros((D, 128), F32).at[:, 2].set(ug_a[:D, 0]),
        jnp.zeros((D, 128), F32).at[:, 3].set(ug_a[D:, 0]),
        jnp.zeros((D, 128), F32).at[:, 4].set(ig_a[:D, 0]),
        jnp.zeros((D, 128), F32).at[:, 5].set(ig_a[D:, 0]),
    ])
    abrow = (jnp.zeros((1, 128), F32)
             .at[0, 0].set(sg_ab[0]).at[0, 2].set(ug_ab[0])
             .at[0, 4].set(ig_ab[0]))
    rpad = jnp.pad(rating_emb, ((0, NR - 10), (0, 0)))
    abot2 = col2(ug_a[D:, 0], ig_a[D:, 0])

    tbl_shape = jax.ShapeDtypeStruct((NC, NP, 128), F32)
    tc1 = pl.pallas_call(
        _tc1_body,
        grid=(GRID,),
        in_specs=[
            pl.BlockSpec((BLK, D), lambda i: (i, 0)),
            pl.BlockSpec((BLK, D), lambda i: (i, 0)),
            pl.BlockSpec((D, D), lambda i: (0, 0)),
            pl.BlockSpec((D, D), lambda i: (0, 0)),
            pl.BlockSpec((D, D), lambda i: (0, 0)),
            pl.BlockSpec((1, D), lambda i: (0, 0)),
            pl.BlockSpec((1, D), lambda i: (0, 0)),
            pl.BlockSpec((1, D), lambda i: (0, 0)),
            pl.BlockSpec((5, D, 128), lambda i: (0, 0, 0)),
            pl.BlockSpec((1, 128), lambda i: (0, 0)),
            pl.BlockSpec((NR, D), lambda i: (0, 0)),
            pl.BlockSpec((D, 128), lambda i: (0, 0)),
        ],
        out_specs=[
            pl.BlockSpec((NC, BLK, 128), lambda i: (0, i, 0)),
            pl.BlockSpec((NC, BLK, 128), lambda i: (0, i, 0)),
            pl.BlockSpec((NC, BLK, 128), lambda i: (0, i, 0)),
            pl.BlockSpec((BLK, 128), lambda i: (i, 0)),
            pl.BlockSpec((NR, 128), lambda i: (0, 0)),
        ],
        out_shape=[tbl_shape, tbl_shape, tbl_shape,
                   jax.ShapeDtypeStruct((NP, 128), F32),
                   jax.ShapeDtypeStruct((NR, 128), F32)],
    )
    hsg2, hiug2, huig2, scal, rout = tc1(
        ue, ie, sg_W, ug_W, ig_W,
        sg_b.reshape(1, D), ug_b.reshape(1, D), ig_b.reshape(1, D),
        apack, abrow, rpad, abot2)

    dsg = scal[:, 0]
    ssg = scal[:, 1]
    dug = scal[:, 2]
    sug = scal[:, 3]
    dig = scal[:, 4]
    sig = scal[:, 5]
    rau = rout[:, 0]
    rai = rout[:, 1]
    z1d = jnp.zeros((WSZ,), F32)

    mesh = plsc.VectorSubcoreMesh(core_axis_name="c", subcore_axis_name="s")
    sc1 = pl.kernel(
        _sc1_body,
        out_type=[tbl_shape, tbl_shape, tbl_shape,
                  jax.ShapeDtypeStruct((WSZ,), F32),
                  jax.ShapeDtypeStruct((WSZ,), F32),
                  jax.ShapeDtypeStruct((NP,), F32)],
        mesh=mesh,
        compiler_params=pltpu.CompilerParams(needs_layout_passes=False),
        scratch_types=[
            pltpu.VMEM_SHARED((NP, 128), F32),
            pltpu.VMEM_SHARED((NP,), F32),
            pltpu.VMEM_SHARED((WSZ,), F32),
            pltpu.VMEM_SHARED((NP,), F32),
            pltpu.VMEM_SHARED((NP,), F32),
            pltpu.VMEM((3, CH), jnp.int32),
            pltpu.VMEM((CH,), F32),
            pltpu.VMEM((CH,), jnp.int32),
            pltpu.VMEM((CH,), F32),
            pltpu.VMEM((CH,), F32),
            pltpu.VMEM((CH, 128), F32),
            pltpu.VMEM((3, CH), jnp.int32),
            pltpu.VMEM((CH,), F32),
            pltpu.VMEM((CH,), jnp.int32),
            pltpu.VMEM((CH,), F32),
            pltpu.VMEM((CH,), F32),
            pltpu.VMEM((CH, 128), F32),
            pltpu.VMEM((16,), F32),
            pltpu.SemaphoreType.DMA,
            pltpu.SemaphoreType.DMA,
            pltpu.SemaphoreType.DMA,
            pltpu.SemaphoreType.DMA,
            pltpu.SemaphoreType.DMA,
            pltpu.SemaphoreType.DMA,
            pltpu.SemaphoreType.DMA,
            pltpu.SemaphoreType.DMA,
        ],
    )
    usagg, uhagg, ihagg, wugf, wigf, den_sg = sc1(
        comb_sg, comb_ug, comb_ig, hsg2, hiug2, huig2,
        dsg, ssg, dug, sug, dig, sig, rau, rai, z1d)

    denp = jnp.pad(den_sg[:, None], ((0, 0), (0, 7)))
    tc2 = pl.pallas_call(
        _tc2_body,
        grid=(GRID,),
        in_specs=[
            pl.BlockSpec((NC, BLK, 128), lambda i: (0, i, 0)),
            pl.BlockSpec((NC, BLK, 128), lambda i: (0, i, 0)),
            pl.BlockSpec((NC, BLK, 128), lambda i: (0, i, 0)),
            pl.BlockSpec((BLK, D), lambda i: (i, 0)),
            pl.BlockSpec((BLK, NR), lambda i: (i, 0)),
            pl.BlockSpec((BLK, NR), lambda i: (i, 0)),
            pl.BlockSpec((BLK, 8), lambda i: (i, 0)),
            pl.BlockSpec((2 * D, D), lambda i: (0, 0)),
            pl.BlockSpec((1, D), lambda i: (0, 0)),
            pl.BlockSpec((2 * D, D), lambda i: (0, 0)),
            pl.BlockSpec((1, D), lambda i: (0, 0)),
            pl.BlockSpec((NR, D), lambda i: (0, 0)),
            pl.BlockSpec((D, 128), lambda i: (0, 0)),
            pl.BlockSpec((D, 128), lambda i: (0, 0)),
            pl.BlockSpec((1, 128), lambda i: (0, 0)),
        ],
        out_specs=[
            pl.BlockSpec((BLK, 128), lambda i: (i, 0)),
            pl.BlockSpec((BLK, 128), lambda i: (i, 0)),
        ],
        out_shape=[jax.ShapeDtypeStruct((NP, 128), F32),
                   jax.ShapeDtypeStruct((NP, 128), F32)],
    )
    Pu, Pi = tc2(usagg, uhagg, ihagg, ie,
                 jnp.pad(wugf.reshape(NP, NRW), ((0, 0), (0, NR - NRW))),
                 jnp.pad(wigf.reshape(NP, NRW), ((0, 0), (0, NR - NRW))), denp,
                 uf_W, uf_b.reshape(1, D), if_W, if_b.reshape(1, D),
                 rpad, jnp.pad(p1_W[:D], ((0, 0), (0, 96))),
                 jnp.pad(p1_W[D:], ((0, 0), (0, 96))),
                 jnp.pad(p1_b.reshape(1, 32), ((0, 0), (0, 96))))

    nb = B // (NC * NT)
    sc2 = pl.kernel(
        _sc2_body,
        out_type=[jax.ShapeDtypeStruct((B,), F32)],
        mesh=mesh,
        compiler_params=pltpu.CompilerParams(needs_layout_passes=False),
        scratch_types=[
            pltpu.VMEM((nb,), jnp.int32),
            pltpu.VMEM((nb,), jnp.int32),
            pltpu.VMEM((nb, 128), F32),
            pltpu.VMEM((nb, 128), F32),
            pltpu.VMEM((32,), F32),
            pltpu.VMEM((16,), F32),
            pltpu.VMEM((nb,), F32),
            pltpu.SemaphoreType.DMA,
        ],
    )
    (score,) = sc2(user_ids, item_ids, Pu, Pi, p2_W.reshape(32),
                   jnp.pad(p2_b, (0, 15)))
    return score


# speedup vs baseline: 1.0052x; 1.0052x over previous
import functools

import jax
import jax.numpy as jnp
from jax import lax
from jax.experimental import pallas as pl
from jax.experimental.pallas import tpu as pltpu
from jax.experimental.pallas import tpu_sc as plsc

N = 10000
NP = 10240
D = 256
E = 160000
B = 4096
NT = 16
NC = 2
CH = 128
EPT_RAW = E // NT
NCHUNK = (EPT_RAW + CH - 1) // CH
EPT = NCHUNK * CH
DUMMY = NP - 1
NR = 16
NRW = 10
WSZ = NP * NRW
RPT = NP // NT
BLK = 256
GRID = NP // BLK
F32 = jnp.float32


def _tc1_body(ue, ie, sgW, ugW, igW, sgb, ugb, igb, apack, abrow, rpad, abot2,
              hsg, hiug, huig, scal, rout):
    i = pl.program_id(0)
    hp = dict(preferred_element_type=F32, precision=lax.Precision.HIGHEST)
    xu = ue[...]
    xi = ie[...]
    hs = jnp.dot(xu, sgW[...], **hp) + sgb[...]
    hui = jnp.dot(xu, ugW[...], **hp) + ugb[...]
    hii = jnp.dot(xi, ugW[...], **hp) + ugb[...]
    hgu = jnp.dot(xu, igW[...], **hp) + igb[...]
    hgi = jnp.dot(xi, igW[...], **hp) + igb[...]
    hsg[0] = hs[:, :128]
    hsg[1] = hs[:, 128:]
    hiug[0] = hii[:, :128]
    hiug[1] = hii[:, 128:]
    huig[0] = hgu[:, :128]
    huig[1] = hgu[:, 128:]
    scal[...] = (jnp.dot(hs, apack[0], **hp) + jnp.dot(hui, apack[1], **hp)
                 + jnp.dot(hii, apack[2], **hp) + jnp.dot(hgi, apack[3], **hp)
                 + jnp.dot(hgu, apack[4], **hp) + abrow[...])

    @pl.when(i == 0)
    def _():
        rout[...] = jnp.dot(rpad[...], abot2[...], **hp)


def _tc2_body(usr, uhr, ihr, ier, wugr, wigr, denr, ufW, ufb, ifW, ifb,
              rpadr, p1a, p1c, p1b, pu, pi):
    hp = dict(preferred_element_type=F32, precision=lax.Precision.HIGHEST)
    dd = denr[...]
    inv_sg = 1.0 / (dd[:, 0:1] + 1e-16)
    inv_ug = 1.0 / (jnp.sum(wugr[...], axis=1, keepdims=True) + 1e-16)
    inv_ig = 1.0 / (jnp.sum(wigr[...], axis=1, keepdims=True) + 1e-16)
    us = jnp.concatenate([usr[0], usr[1]], axis=1) * inv_sg
    uh = (jnp.concatenate([uhr[0], uhr[1]], axis=1)
          + jnp.dot(wugr[...], rpadr[...], **hp)) * inv_ug
    ucat = jnp.concatenate([us, uh], axis=1)
    fu = jnp.maximum(jnp.dot(ucat, ufW[...], **hp) + ufb[...], 0.0)
    pu[...] = jnp.dot(fu, p1a[...], **hp) + p1b[...]
    ih = (jnp.concatenate([ihr[0], ihr[1]], axis=1)
          + jnp.dot(wigr[...], rpadr[...], **hp)) * inv_ig
    icat = jnp.concatenate([ier[...], ih], axis=1)
    fi = jnp.maximum(jnp.dot(icat, ifW[...], **hp) + ifb[...], 0.0)
    pi[...] = jnp.dot(fi, p1c[...], **hp)


def _sc1_body(comb_sg, comb_ug, comb_ig,
              hsg2, hiug2, huig2,
              dsg, ssg, dug, sug, dig, sig, rau, rai, z1d,
              usagg, uhagg, ihagg, wug, wig, dnsg,
              acc, den, wsp, dsp, ssp, ebuf, exb, wix, dgb, sgb, rbuf,
              rtt, sem, sem2, sem3):
    c = lax.axis_index("c")
    t = lax.axis_index("s")
    r0 = t * RPT

    def run_gat(comb, d_hbm, s_hbm, tbl, ratt_hbm, out_hbm, den_hbm, w_hbm,
                use_rating, aux_core):
        def zrow(r, carry2):
            for m in range(8):
                rbuf[r, pl.ds(m * 16, 16)] = jnp.zeros((16,), F32)
            return carry2

        lax.fori_loop(0, CH, zrow, 0)
        for kk in range(RPT // CH):
            pltpu.sync_copy(rbuf, acc.at[pl.ds(r0 + kk * CH, CH)])
        if use_rating:
            pltpu.sync_copy(z1d.at[pl.ds(t * RPT * NRW, RPT * NRW)],
                            wsp.at[pl.ds(t * RPT * NRW, RPT * NRW)])
            pltpu.sync_copy(ratt_hbm, rtt)
        else:
            pltpu.sync_copy(z1d.at[pl.ds(r0, RPT)], den.at[pl.ds(r0, RPT)])
        pltpu.sync_copy(d_hbm.at[pl.ds(r0, RPT)], dsp.at[pl.ds(r0, RPT)])
        pltpu.sync_copy(s_hbm.at[pl.ds(r0, RPT)], ssp.at[pl.ds(r0, RPT)])
        plsc.subcore_barrier()

        def chunk(j, carry):
            pltpu.sync_copy(comb.at[t, j], ebuf)
            cpd = pltpu.async_copy(dsp.at[ebuf.at[1]], dgb, sem2)
            cps = pltpu.async_copy(ssp.at[ebuf.at[0]], sgb, sem3)
            cpr = pltpu.async_copy(tbl.at[c].at[ebuf.at[0]], rbuf, sem)
            cpd.wait()
            cps.wait()
            for k in range(CH // 16):
                sl = pl.ds(k * 16, 16)
                e = dgb[sl] + sgb[sl]
                if use_rating:
                    rr = plsc.bitcast(ebuf[2, sl], F32)
                    ri = jnp.clip((rr * 2.0 - 1.0).astype(jnp.int32), 0, 9)
                    e = e + plsc.load_gather(rtt, [ri])
                    wix[sl] = ebuf[1, sl] * NRW + ri
                e = jnp.where(e >= 0.0, e, 0.2 * e)
                exb[sl] = jnp.exp(e)
            @pl.when(c == aux_core)
            def _():
                if use_rating:
                    pltpu.sync_copy(exb, wsp.at[wix], add=True)
                else:
                    pltpu.sync_copy(exb, den.at[ebuf.at[1]], add=True)
            cpr.wait()

            def rows(g, carry2):
                for h in range(4):
                    exv = exb[pl.ds(g * 64 + h * 16, 16)]
                    for q in range(16):
                        s = exv[q]
                        r = g * 64 + h * 16 + q
                        for m in range(8):
                            msl = pl.ds(m * 16, 16)
                            rbuf[r, msl] = rbuf[r, msl] * s
                return carry2

            lax.fori_loop(0, CH // 64, rows, 0)
            pltpu.sync_copy(rbuf, acc.at[ebuf.at[1]], add=True)
            return carry

        lax.fori_loop(0, NCHUNK, chunk, 0)
        plsc.subcore_barrier()

        pltpu.sync_copy(acc.at[pl.ds(r0, RPT)], out_hbm.at[c, pl.ds(r0, RPT)])
        if use_rating:
            @pl.when(c == aux_core)
            def _():
                pltpu.sync_copy(wsp.at[pl.ds(t * RPT * NRW, RPT * NRW)],
                                w_hbm.at[pl.ds(t * RPT * NRW, RPT * NRW)])
        else:
            @pl.when(c == aux_core)
            def _():
                pltpu.sync_copy(den.at[pl.ds(r0, RPT)],
                                den_hbm.at[pl.ds(r0, RPT)])

    run_gat(comb_sg, dsg, ssg, hsg2, None, usagg, dnsg, None, False, 1)
    run_gat(comb_ug, dug, sug, hiug2, rau, uhagg, None, wug, True, 0)
    run_gat(comb_ig, dig, sig, huig2, rai, ihagg, None, wig, True, 1)


def _sc2_body(uid, iid, Pu, Pi, p2v, p2s, score, uv, iv, pur, pir, p2b, p2bs,
              ob, sem):
    c = lax.axis_index("c")
    t = lax.axis_index("s")
    w = t * NC + c
    nb = B // (NC * NT)
    base = w * nb
    pltpu.sync_copy(uid.at[pl.ds(base, nb)], uv)
    pltpu.sync_copy(iid.at[pl.ds(base, nb)], iv)
    pltpu.async_copy(Pu.at[uv], pur, sem).wait()
    pltpu.async_copy(Pi.at[iv], pir, sem).wait()
    pltpu.sync_copy(p2v, p2b)
    pltpu.sync_copy(p2s, p2bs)
    p2lo = p2b[pl.ds(0, 16)]
    p2hi = p2b[pl.ds(16, 16)]
    bias = p2bs[...][0]

    def grp(g, carry):
        rows16 = lax.iota(jnp.int32, 16) + g * 16
        acc16 = jnp.zeros((16,), F32)
        for j in range(32):
            cj = jnp.full((16,), j, jnp.int32)
            v = (plsc.load_gather(pur, [rows16, cj])
                 + plsc.load_gather(pir, [rows16, cj]))
            p2j = p2lo[j] if j < 16 else p2hi[j - 16]
            acc16 = acc16 + jnp.maximum(v, 0.0) * p2j
        ob[pl.ds(g * 16, 16)] = acc16 + bias
        return carry

    lax.fori_loop(0, nb // 16, grp, 0)
    pltpu.sync_copy(ob, score.at[pl.ds(base, nb)])


def kernel(user_ids, item_ids, social_adj, interact_adj, interact_ratings,
           user_emb, item_emb, rating_emb,
           sg_W, sg_b, sg_a, sg_ab,
           ug_W, ug_b, ug_a, ug_ab,
           ig_W, ig_b, ig_a, ig_ab,
           uf_W, uf_b, if_W, if_b,
           p1_W, p1_b, p2_W, p2_b):
    ue = jnp.pad(user_emb, ((0, NP - N), (0, 0)))
    ie = jnp.pad(item_emb, ((0, NP - N), (0, 0)))

    def prep_idx(x, fill):
        x = x.reshape(NT, EPT_RAW)
        x = jnp.pad(x, ((0, 0), (0, EPT - EPT_RAW)), constant_values=fill)
        return x.reshape(NT, NCHUNK, CH)

    sa0p = prep_idx(social_adj[0], DUMMY)
    sa1p = prep_idx(social_adj[1], DUMMY)
    ia0p = prep_idx(interact_adj[0], DUMMY)
    ia1p = prep_idx(interact_adj[1], DUMMY)
    ratb = lax.bitcast_convert_type(prep_idx(interact_ratings, 1.0),
                                    jnp.int32)
    comb_sg = jnp.stack([sa0p, sa1p, jnp.zeros_like(sa0p)], axis=2)
    comb_ug = jnp.stack([ia1p, ia0p, ratb], axis=2)
    comb_ig = jnp.stack([ia0p, ia1p, ratb], axis=2)

    def col2(v0, v1):
        z = jnp.zeros((D, 128), F32)
        return z.at[:, 0].set(v0).at[:, 1].set(v1)

    apack = jnp.stack([
        col2(sg_a[:D, 0], sg_a[D:, 0]),
        jnp.zeros((D, 128), F32).at[:, 2].set(ug_a[:D, 0]),
        jnp.zeros((D, 128), F32).at[:, 3].set(ug_a[D:, 0]),
        jnp.zeros((D, 128), F32).at[:, 4].set(ig_a[:D, 0]),
        jnp.zeros((D, 128), F32).at[:, 5].set(ig_a[D:, 0]),
    ])
    abrow = (jnp.zeros((1, 128), F32)
             .at[0, 0].set(sg_ab[0]).at[0, 2].set(ug_ab[0])
             .at[0, 4].set(ig_ab[0]))
    rpad = jnp.pad(rating_emb, ((0, NR - 10), (0, 0)))
    abot2 = col2(ug_a[D:, 0], ig_a[D:, 0])

    tbl_shape = jax.ShapeDtypeStruct((NC, NP, 128), F32)
    tc1 = pl.pallas_call(
        _tc1_body,
        grid=(GRID,),
        in_specs=[
            pl.BlockSpec((BLK, D), lambda i: (i, 0)),
            pl.BlockSpec((BLK, D), lambda i: (i, 0)),
            pl.BlockSpec((D, D), lambda i: (0, 0)),
            pl.BlockSpec((D, D), lambda i: (0, 0)),
            pl.BlockSpec((D, D), lambda i: (0, 0)),
            pl.BlockSpec((1, D), lambda i: (0, 0)),
            pl.BlockSpec((1, D), lambda i: (0, 0)),
            pl.BlockSpec((1, D), lambda i: (0, 0)),
            pl.BlockSpec((5, D, 128), lambda i: (0, 0, 0)),
            pl.BlockSpec((1, 128), lambda i: (0, 0)),
            pl.BlockSpec((NR, D), lambda i: (0, 0)),
            pl.BlockSpec((D, 128), lambda i: (0, 0)),
        ],
        out_specs=[
            pl.BlockSpec((NC, BLK, 128), lambda i: (0, i, 0)),
            pl.BlockSpec((NC, BLK, 128), lambda i: (0, i, 0)),
            pl.BlockSpec((NC, BLK, 128), lambda i: (0, i, 0)),
            pl.BlockSpec((BLK, 128), lambda i: (i, 0)),
            pl.BlockSpec((NR, 128), lambda i: (0, 0)),
        ],
        out_shape=[tbl_shape, tbl_shape, tbl_shape,
                   jax.ShapeDtypeStruct((NP, 128), F32),
                   jax.ShapeDtypeStruct((NR, 128), F32)],
    )
    hsg2, hiug2, huig2, scal, rout = tc1(
        ue, ie, sg_W, ug_W, ig_W,
        sg_b.reshape(1, D), ug_b.reshape(1, D), ig_b.reshape(1, D),
        apack, abrow, rpad, abot2)

    dsg = scal[:, 0]
    ssg = scal[:, 1]
    dug = scal[:, 2]
    sug = scal[:, 3]
    dig = scal[:, 4]
    sig = scal[:, 5]
    rau = rout[:, 0]
    rai = rout[:, 1]
    z1d = jnp.zeros((WSZ,), F32)

    mesh = plsc.VectorSubcoreMesh(core_axis_name="c", subcore_axis_name="s")
    sc1 = pl.kernel(
        _sc1_body,
        out_type=[tbl_shape, tbl_shape, tbl_shape,
                  jax.ShapeDtypeStruct((WSZ,), F32),
                  jax.ShapeDtypeStruct((WSZ,), F32),
                  jax.ShapeDtypeStruct((NP,), F32)],
        mesh=mesh,
        compiler_params=pltpu.CompilerParams(needs_layout_passes=False),
        scratch_types=[
            pltpu.VMEM_SHARED((NP, 128), F32),
            pltpu.VMEM_SHARED((NP,), F32),
            pltpu.VMEM_SHARED((WSZ,), F32),
            pltpu.VMEM_SHARED((NP,), F32),
            pltpu.VMEM_SHARED((NP,), F32),
            pltpu.VMEM((3, CH), jnp.int32),
            pltpu.VMEM((CH,), F32),
            pltpu.VMEM((CH,), jnp.int32),
            pltpu.VMEM((CH,), F32),
            pltpu.VMEM((CH,), F32),
            pltpu.VMEM((CH, 128), F32),
            pltpu.VMEM((16,), F32),
            pltpu.SemaphoreType.DMA,
            pltpu.SemaphoreType.DMA,
            pltpu.SemaphoreType.DMA,
        ],
    )
    usagg, uhagg, ihagg, wugf, wigf, den_sg = sc1(
        comb_sg, comb_ug, comb_ig, hsg2, hiug2, huig2,
        dsg, ssg, dug, sug, dig, sig, rau, rai, z1d)

    denp = jnp.pad(den_sg[:, None], ((0, 0), (0, 7)))
    tc2 = pl.pallas_call(
        _tc2_body,
        grid=(GRID,),
        in_specs=[
            pl.BlockSpec((NC, BLK, 128), lambda i: (0, i, 0)),
            pl.BlockSpec((NC, BLK, 128), lambda i: (0, i, 0)),
            pl.BlockSpec((NC, BLK, 128), lambda i: (0, i, 0)),
            pl.BlockSpec((BLK, D), lambda i: (i, 0)),
            pl.BlockSpec((BLK, NR), lambda i: (i, 0)),
            pl.BlockSpec((BLK, NR), lambda i: (i, 0)),
            pl.BlockSpec((BLK, 8), lambda i: (i, 0)),
            pl.BlockSpec((2 * D, D), lambda i: (0, 0)),
            pl.BlockSpec((1, D), lambda i: (0, 0)),
            pl.BlockSpec((2 * D, D), lambda i: (0, 0)),
            pl.BlockSpec((1, D), lambda i: (0, 0)),
            pl.BlockSpec((NR, D), lambda i: (0, 0)),
            pl.BlockSpec((D, 128), lambda i: (0, 0)),
            pl.BlockSpec((D, 128), lambda i: (0, 0)),
            pl.BlockSpec((1, 128), lambda i: (0, 0)),
        ],
        out_specs=[
            pl.BlockSpec((BLK, 128), lambda i: (i, 0)),
            pl.BlockSpec((BLK, 128), lambda i: (i, 0)),
        ],
        out_shape=[jax.ShapeDtypeStruct((NP, 128), F32),
                   jax.ShapeDtypeStruct((NP, 128), F32)],
    )
    Pu, Pi = tc2(usagg, uhagg, ihagg, ie,
                 jnp.pad(wugf.reshape(NP, NRW), ((0, 0), (0, NR - NRW))),
                 jnp.pad(wigf.reshape(NP, NRW), ((0, 0), (0, NR - NRW))), denp,
                 uf_W, uf_b.reshape(1, D), if_W, if_b.reshape(1, D),
                 rpad, jnp.pad(p1_W[:D], ((0, 0), (0, 96))),
                 jnp.pad(p1_W[D:], ((0, 0), (0, 96))),
                 jnp.pad(p1_b.reshape(1, 32), ((0, 0), (0, 96))))

    nb = B // (NC * NT)
    sc2 = pl.kernel(
        _sc2_body,
        out_type=[jax.ShapeDtypeStruct((B,), F32)],
        mesh=mesh,
        compiler_params=pltpu.CompilerParams(needs_layout_passes=False),
        scratch_types=[
            pltpu.VMEM((nb,), jnp.int32),
            pltpu.VMEM((nb,), jnp.int32),
            pltpu.VMEM((nb, 128), F32),
            pltpu.VMEM((nb, 128), F32),
            pltpu.VMEM((32,), F32),
            pltpu.VMEM((16,), F32),
            pltpu.VMEM((nb,), F32),
            pltpu.SemaphoreType.DMA,
        ],
    )
    (score,) = sc2(user_ids, item_ids, Pu, Pi, p2_W.reshape(32),
                   jnp.pad(p2_b, (0, 15)))
    return score


# speedup vs baseline: 1.1155x; 1.1098x over previous
import functools

import jax
import jax.numpy as jnp
from jax import lax
from jax.experimental import pallas as pl
from jax.experimental.pallas import tpu as pltpu
from jax.experimental.pallas import tpu_sc as plsc

N = 10000
NP = 10240
D = 256
E = 160000
B = 4096
NT = 16
NC = 2
CH = 128
EPT_RAW = E // NT
NCHUNK = (EPT_RAW + CH - 1) // CH
EPT = NCHUNK * CH
DUMMY = NP - 1
NR = 16
NRW = 10
WSZ = NP * NRW
RPT = NP // NT
BLK = 256
GRID = NP // BLK
F32 = jnp.float32


def _tc1_body(ue, ie, sgW, ugW, igW, sgb, ugb, igb, apack, abrow, rpad, abot2,
              hsg, hiug, huig, scal, rout):
    i = pl.program_id(0)
    hp = dict(preferred_element_type=F32)
    xu = ue[...]
    xi = ie[...]
    hs = jnp.dot(xu, sgW[...], **hp) + sgb[...]
    hui = jnp.dot(xu, ugW[...], **hp) + ugb[...]
    hii = jnp.dot(xi, ugW[...], **hp) + ugb[...]
    hgu = jnp.dot(xu, igW[...], **hp) + igb[...]
    hgi = jnp.dot(xi, igW[...], **hp) + igb[...]
    hsg[0] = hs[:, :128]
    hsg[1] = hs[:, 128:]
    hiug[0] = hii[:, :128]
    hiug[1] = hii[:, 128:]
    huig[0] = hgu[:, :128]
    huig[1] = hgu[:, 128:]
    scal[...] = (jnp.dot(hs, apack[0], **hp) + jnp.dot(hui, apack[1], **hp)
                 + jnp.dot(hii, apack[2], **hp) + jnp.dot(hgi, apack[3], **hp)
                 + jnp.dot(hgu, apack[4], **hp) + abrow[...])

    @pl.when(i == 0)
    def _():
        rout[...] = jnp.dot(rpad[...], abot2[...], **hp)


def _tc2_body(usr, uhr, ihr, ier, wugr, wigr, denr, ufW, ufb, ifW, ifb,
              rpadr, p1a, p1c, p1b, pu, pi):
    hp = dict(preferred_element_type=F32)
    dd = denr[...]
    inv_sg = 1.0 / (dd[:, 0:1] + 1e-16)
    inv_ug = 1.0 / (jnp.sum(wugr[...], axis=1, keepdims=True) + 1e-16)
    inv_ig = 1.0 / (jnp.sum(wigr[...], axis=1, keepdims=True) + 1e-16)
    us = jnp.concatenate([usr[0], usr[1]], axis=1) * inv_sg
    uh = (jnp.concatenate([uhr[0], uhr[1]], axis=1)
          + jnp.dot(wugr[...], rpadr[...], **hp)) * inv_ug
    ucat = jnp.concatenate([us, uh], axis=1)
    fu = jnp.maximum(jnp.dot(ucat, ufW[...], **hp) + ufb[...], 0.0)
    pu[...] = jnp.dot(fu, p1a[...], **hp) + p1b[...]
    ih = (jnp.concatenate([ihr[0], ihr[1]], axis=1)
          + jnp.dot(wigr[...], rpadr[...], **hp)) * inv_ig
    icat = jnp.concatenate([ier[...], ih], axis=1)
    fi = jnp.maximum(jnp.dot(icat, ifW[...], **hp) + ifb[...], 0.0)
    pi[...] = jnp.dot(fi, p1c[...], **hp)


def _sc1_body(comb_sg, comb_ug, comb_ig,
              hsg2, hiug2, huig2,
              dsg, ssg, dug, sug, dig, sig, rau, rai, z1d,
              usagg, uhagg, ihagg, wug, wig, dnsg,
              acc, den, wsp, dsp, ssp, ebuf, exb, wix, dgb, sgb, rbuf,
              rtt, sem, sem2, sem3):
    c = lax.axis_index("c")
    t = lax.axis_index("s")
    r0 = t * RPT

    def run_gat(comb, d_hbm, s_hbm, tbl, ratt_hbm, out_hbm, den_hbm, w_hbm,
                use_rating, aux_core):
        def zrow(r, carry2):
            for m in range(8):
                rbuf[r, pl.ds(m * 16, 16)] = jnp.zeros((16,), F32)
            return carry2

        lax.fori_loop(0, CH, zrow, 0)
        for kk in range(RPT // CH):
            pltpu.sync_copy(rbuf, acc.at[pl.ds(r0 + kk * CH, CH)])
        if use_rating:
            pltpu.sync_copy(z1d.at[pl.ds(t * RPT * NRW, RPT * NRW)],
                            wsp.at[pl.ds(t * RPT * NRW, RPT * NRW)])
            pltpu.sync_copy(ratt_hbm, rtt)
        else:
            pltpu.sync_copy(z1d.at[pl.ds(r0, RPT)], den.at[pl.ds(r0, RPT)])
        pltpu.sync_copy(d_hbm.at[pl.ds(r0, RPT)], dsp.at[pl.ds(r0, RPT)])
        pltpu.sync_copy(s_hbm.at[pl.ds(r0, RPT)], ssp.at[pl.ds(r0, RPT)])
        plsc.subcore_barrier()

        def chunk(j, carry):
            pltpu.sync_copy(comb.at[t, j], ebuf)
            cpd = pltpu.async_copy(dsp.at[ebuf.at[1]], dgb, sem2)
            cps = pltpu.async_copy(ssp.at[ebuf.at[0]], sgb, sem3)
            cpr = pltpu.async_copy(tbl.at[c].at[ebuf.at[0]], rbuf, sem)
            cpd.wait()
            cps.wait()
            for k in range(CH // 16):
                sl = pl.ds(k * 16, 16)
                e = dgb[sl] + sgb[sl]
                if use_rating:
                    rr = plsc.bitcast(ebuf[2, sl], F32)
                    ri = jnp.clip((rr * 2.0 - 1.0).astype(jnp.int32), 0, 9)
                    e = e + plsc.load_gather(rtt, [ri])
                    wix[sl] = ebuf[1, sl] * NRW + ri
                e = jnp.where(e >= 0.0, e, 0.2 * e)
                exb[sl] = jnp.exp(e)
            @pl.when(c == aux_core)
            def _():
                if use_rating:
                    pltpu.sync_copy(exb, wsp.at[wix], add=True)
                else:
                    pltpu.sync_copy(exb, den.at[ebuf.at[1]], add=True)
            cpr.wait()

            def rows(g, carry2):
                for h in range(4):
                    exv = exb[pl.ds(g * 64 + h * 16, 16)]
                    for q in range(16):
                        s = exv[q]
                        r = g * 64 + h * 16 + q
                        for m in range(8):
                            msl = pl.ds(m * 16, 16)
                            rbuf[r, msl] = rbuf[r, msl] * s
                return carry2

            lax.fori_loop(0, CH // 64, rows, 0)
            pltpu.sync_copy(rbuf, acc.at[ebuf.at[1]], add=True)
            return carry

        lax.fori_loop(0, NCHUNK, chunk, 0)
        plsc.subcore_barrier()

        pltpu.sync_copy(acc.at[pl.ds(r0, RPT)], out_hbm.at[c, pl.ds(r0, RPT)])
        if use_rating:
            @pl.when(c == aux_core)
            def _():
                pltpu.sync_copy(wsp.at[pl.ds(t * RPT * NRW, RPT * NRW)],
                                w_hbm.at[pl.ds(t * RPT * NRW, RPT * NRW)])
        else:
            @pl.when(c == aux_core)
            def _():
                pltpu.sync_copy(den.at[pl.ds(r0, RPT)],
                                den_hbm.at[pl.ds(r0, RPT)])

    run_gat(comb_sg, dsg, ssg, hsg2, None, usagg, dnsg, None, False, 1)
    run_gat(comb_ug, dug, sug, hiug2, rau, uhagg, None, wug, True, 0)
    run_gat(comb_ig, dig, sig, huig2, rai, ihagg, None, wig, True, 1)


def _sc2_body(uid, iid, Pu, Pi, p2v, p2s, score, uv, iv, pur, pir, p2b, p2bs,
              ob, sem):
    c = lax.axis_index("c")
    t = lax.axis_index("s")
    w = t * NC + c
    nb = B // (NC * NT)
    base = w * nb
    pltpu.sync_copy(uid.at[pl.ds(base, nb)], uv)
    pltpu.sync_copy(iid.at[pl.ds(base, nb)], iv)
    pltpu.async_copy(Pu.at[uv], pur, sem).wait()
    pltpu.async_copy(Pi.at[iv], pir, sem).wait()
    pltpu.sync_copy(p2v, p2b)
    pltpu.sync_copy(p2s, p2bs)
    p2lo = p2b[pl.ds(0, 16)]
    p2hi = p2b[pl.ds(16, 16)]
    bias = p2bs[...][0]

    def grp(g, carry):
        rows16 = lax.iota(jnp.int32, 16) + g * 16
        acc16 = jnp.zeros((16,), F32)
        for j in range(32):
            cj = jnp.full((16,), j, jnp.int32)
            v = (plsc.load_gather(pur, [rows16, cj])
                 + plsc.load_gather(pir, [rows16, cj]))
            p2j = p2lo[j] if j < 16 else p2hi[j - 16]
            acc16 = acc16 + jnp.maximum(v, 0.0) * p2j
        ob[pl.ds(g * 16, 16)] = acc16 + bias
        return carry

    lax.fori_loop(0, nb // 16, grp, 0)
    pltpu.sync_copy(ob, score.at[pl.ds(base, nb)])


def kernel(user_ids, item_ids, social_adj, interact_adj, interact_ratings,
           user_emb, item_emb, rating_emb,
           sg_W, sg_b, sg_a, sg_ab,
           ug_W, ug_b, ug_a, ug_ab,
           ig_W, ig_b, ig_a, ig_ab,
           uf_W, uf_b, if_W, if_b,
           p1_W, p1_b, p2_W, p2_b):
    ue = jnp.pad(user_emb, ((0, NP - N), (0, 0)))
    ie = jnp.pad(item_emb, ((0, NP - N), (0, 0)))

    def prep_idx(x, fill):
        x = x.reshape(NT, EPT_RAW)
        x = jnp.pad(x, ((0, 0), (0, EPT - EPT_RAW)), constant_values=fill)
        return x.reshape(NT, NCHUNK, CH)

    sa0p = prep_idx(social_adj[0], DUMMY)
    sa1p = prep_idx(social_adj[1], DUMMY)
    ia0p = prep_idx(interact_adj[0], DUMMY)
    ia1p = prep_idx(interact_adj[1], DUMMY)
    ratb = lax.bitcast_convert_type(prep_idx(interact_ratings, 1.0),
                                    jnp.int32)
    comb_sg = jnp.stack([sa0p, sa1p, jnp.zeros_like(sa0p)], axis=2)
    comb_ug = jnp.stack([ia1p, ia0p, ratb], axis=2)
    comb_ig = jnp.stack([ia0p, ia1p, ratb], axis=2)

    def col2(v0, v1):
        z = jnp.zeros((D, 128), F32)
        return z.at[:, 0].set(v0).at[:, 1].set(v1)

    apack = jnp.stack([
        col2(sg_a[:D, 0], sg_a[D:, 0]),
        jnp.zeros((D, 128), F32).at[:, 2].set(ug_a[:D, 0]),
        jnp.zeros((D, 128), F32).at[:, 3].set(ug_a[D:, 0]),
        jnp.zeros((D, 128), F32).at[:, 4].set(ig_a[:D, 0]),
        jnp.zeros((D, 128), F32).at[:, 5].set(ig_a[D:, 0]),
    ])
    abrow = (jnp.zeros((1, 128), F32)
             .at[0, 0].set(sg_ab[0]).at[0, 2].set(ug_ab[0])
             .at[0, 4].set(ig_ab[0]))
    rpad = jnp.pad(rating_emb, ((0, NR - 10), (0, 0)))
    abot2 = col2(ug_a[D:, 0], ig_a[D:, 0])

    tbl_shape = jax.ShapeDtypeStruct((NC, NP, 128), F32)
    tc1 = pl.pallas_call(
        _tc1_body,
        grid=(GRID,),
        in_specs=[
            pl.BlockSpec((BLK, D), lambda i: (i, 0)),
            pl.BlockSpec((BLK, D), lambda i: (i, 0)),
            pl.BlockSpec((D, D), lambda i: (0, 0)),
            pl.BlockSpec((D, D), lambda i: (0, 0)),
            pl.BlockSpec((D, D), lambda i: (0, 0)),
            pl.BlockSpec((1, D), lambda i: (0, 0)),
            pl.BlockSpec((1, D), lambda i: (0, 0)),
            pl.BlockSpec((1, D), lambda i: (0, 0)),
            pl.BlockSpec((5, D, 128), lambda i: (0, 0, 0)),
            pl.BlockSpec((1, 128), lambda i: (0, 0)),
            pl.BlockSpec((NR, D), lambda i: (0, 0)),
            pl.BlockSpec((D, 128), lambda i: (0, 0)),
        ],
        out_specs=[
            pl.BlockSpec((NC, BLK, 128), lambda i: (0, i, 0)),
            pl.BlockSpec((NC, BLK, 128), lambda i: (0, i, 0)),
            pl.BlockSpec((NC, BLK, 128), lambda i: (0, i, 0)),
            pl.BlockSpec((BLK, 128), lambda i: (i, 0)),
            pl.BlockSpec((NR, 128), lambda i: (0, 0)),
        ],
        out_shape=[tbl_shape, tbl_shape, tbl_shape,
                   jax.ShapeDtypeStruct((NP, 128), F32),
                   jax.ShapeDtypeStruct((NR, 128), F32)],
    )
    hsg2, hiug2, huig2, scal, rout = tc1(
        ue, ie, sg_W, ug_W, ig_W,
        sg_b.reshape(1, D), ug_b.reshape(1, D), ig_b.reshape(1, D),
        apack, abrow, rpad, abot2)

    dsg = scal[:, 0]
    ssg = scal[:, 1]
    dug = scal[:, 2]
    sug = scal[:, 3]
    dig = scal[:, 4]
    sig = scal[:, 5]
    rau = rout[:, 0]
    rai = rout[:, 1]
    z1d = jnp.zeros((WSZ,), F32)

    mesh = plsc.VectorSubcoreMesh(core_axis_name="c", subcore_axis_name="s")
    sc1 = pl.kernel(
        _sc1_body,
        out_type=[tbl_shape, tbl_shape, tbl_shape,
                  jax.ShapeDtypeStruct((WSZ,), F32),
                  jax.ShapeDtypeStruct((WSZ,), F32),
                  jax.ShapeDtypeStruct((NP,), F32)],
        mesh=mesh,
        compiler_params=pltpu.CompilerParams(needs_layout_passes=False),
        scratch_types=[
            pltpu.VMEM_SHARED((NP, 128), F32),
            pltpu.VMEM_SHARED((NP,), F32),
            pltpu.VMEM_SHARED((WSZ,), F32),
            pltpu.VMEM_SHARED((NP,), F32),
            pltpu.VMEM_SHARED((NP,), F32),
            pltpu.VMEM((3, CH), jnp.int32),
            pltpu.VMEM((CH,), F32),
            pltpu.VMEM((CH,), jnp.int32),
            pltpu.VMEM((CH,), F32),
            pltpu.VMEM((CH,), F32),
            pltpu.VMEM((CH, 128), F32),
            pltpu.VMEM((16,), F32),
            pltpu.SemaphoreType.DMA,
            pltpu.SemaphoreType.DMA,
            pltpu.SemaphoreType.DMA,
        ],
    )
    usagg, uhagg, ihagg, wugf, wigf, den_sg = sc1(
        comb_sg, comb_ug, comb_ig, hsg2, hiug2, huig2,
        dsg, ssg, dug, sug, dig, sig, rau, rai, z1d)

    denp = jnp.pad(den_sg[:, None], ((0, 0), (0, 7)))
    tc2 = pl.pallas_call(
        _tc2_body,
        grid=(GRID,),
        in_specs=[
            pl.BlockSpec((NC, BLK, 128), lambda i: (0, i, 0)),
            pl.BlockSpec((NC, BLK, 128), lambda i: (0, i, 0)),
            pl.BlockSpec((NC, BLK, 128), lambda i: (0, i, 0)),
            pl.BlockSpec((BLK, D), lambda i: (i, 0)),
            pl.BlockSpec((BLK, NR), lambda i: (i, 0)),
            pl.BlockSpec((BLK, NR), lambda i: (i, 0)),
            pl.BlockSpec((BLK, 8), lambda i: (i, 0)),
            pl.BlockSpec((2 * D, D), lambda i: (0, 0)),
            pl.BlockSpec((1, D), lambda i: (0, 0)),
            pl.BlockSpec((2 * D, D), lambda i: (0, 0)),
            pl.BlockSpec((1, D), lambda i: (0, 0)),
            pl.BlockSpec((NR, D), lambda i: (0, 0)),
            pl.BlockSpec((D, 128), lambda i: (0, 0)),
            pl.BlockSpec((D, 128), lambda i: (0, 0)),
            pl.BlockSpec((1, 128), lambda i: (0, 0)),
        ],
        out_specs=[
            pl.BlockSpec((BLK, 128), lambda i: (i, 0)),
            pl.BlockSpec((BLK, 128), lambda i: (i, 0)),
        ],
        out_shape=[jax.ShapeDtypeStruct((NP, 128), F32),
                   jax.ShapeDtypeStruct((NP, 128), F32)],
    )
    Pu, Pi = tc2(usagg, uhagg, ihagg, ie,
                 jnp.pad(wugf.reshape(NP, NRW), ((0, 0), (0, NR - NRW))),
                 jnp.pad(wigf.reshape(NP, NRW), ((0, 0), (0, NR - NRW))), denp,
                 uf_W, uf_b.reshape(1, D), if_W, if_b.reshape(1, D),
                 rpad, jnp.pad(p1_W[:D], ((0, 0), (0, 96))),
                 jnp.pad(p1_W[D:], ((0, 0), (0, 96))),
                 jnp.pad(p1_b.reshape(1, 32), ((0, 0), (0, 96))))

    nb = B // (NC * NT)
    sc2 = pl.kernel(
        _sc2_body,
        out_type=[jax.ShapeDtypeStruct((B,), F32)],
        mesh=mesh,
        compiler_params=pltpu.CompilerParams(needs_layout_passes=False),
        scratch_types=[
            pltpu.VMEM((nb,), jnp.int32),
            pltpu.VMEM((nb,), jnp.int32),
            pltpu.VMEM((nb, 128), F32),
            pltpu.VMEM((nb, 128), F32),
            pltpu.VMEM((32,), F32),
            pltpu.VMEM((16,), F32),
            pltpu.VMEM((nb,), F32),
            pltpu.SemaphoreType.DMA,
        ],
    )
    (score,) = sc2(user_ids, item_ids, Pu, Pi, p2_W.reshape(32),
                   jnp.pad(p2_b, (0, 15)))
    return score
